# baseline (speedup 1.0000x reference)
import functools

import jax
import jax.numpy as jnp
from jax import lax
from jax.experimental import pallas as pl
from jax.experimental.pallas import tpu as pltpu
from jax.experimental.pallas import tpu_sc as plsc

NUM_USERS = 1000000
NUM_ITEMS = 100000
EMBED_DIM = 32
BATCH = 16384

NC = 2
NS = 16
NW = NC * NS
BPW = BATCH // NW
CHUNK = 128
NCHUNK = BPW // CHUNK
GROUPS = BPW // 16

_mesh = plsc.VectorSubcoreMesh(core_axis_name="c", subcore_axis_name="s")


@functools.partial(
    pl.kernel,
    out_type=jax.ShapeDtypeStruct((BATCH,), jnp.float32),
    mesh=_mesh,
    compiler_params=pltpu.CompilerParams(needs_layout_passes=False,
                                         use_tc_tiling_on_sc=False),
    scratch_types=[
        pltpu.VMEM((NCHUNK, CHUNK), jnp.int32),
        pltpu.VMEM((NCHUNK, CHUNK), jnp.int32),
        pltpu.VMEM((BPW, EMBED_DIM), jnp.float32),
        pltpu.VMEM((BPW, EMBED_DIM), jnp.float32),
        pltpu.VMEM((BPW,), jnp.float32),
        pltpu.VMEM((BPW,), jnp.float32),
        pltpu.VMEM((16,), jnp.float32),
        pltpu.VMEM((BPW,), jnp.float32),
        pltpu.SemaphoreType.DMA,
    ],
)
def _mf_sc(uids_hbm, iids_hbm, utab_hbm, itab_hbm, ub_hbm, ib_hbm, gb_hbm,
           out_hbm, uidx_v, iidx_v, urows_v, irows_v, ubv_v, ibv_v, gb_v,
           out_v, sem):
    wid = lax.axis_index("s") * NC + lax.axis_index("c")
    base = pl.multiple_of(wid * BPW, BPW)

    pltpu.sync_copy(uids_hbm.at[wid], uidx_v)
    pltpu.sync_copy(iids_hbm.at[wid], iidx_v)
    pltpu.sync_copy(gb_hbm, gb_v)

    cps = []
    for j in range(NCHUNK):
        dst = pl.ds(j * CHUNK, CHUNK)
        cps.append(pltpu.async_copy(utab_hbm.at[uidx_v.at[j]],
                                    urows_v.at[dst], sem))
        cps.append(pltpu.async_copy(itab_hbm.at[iidx_v.at[j]],
                                    irows_v.at[dst], sem))
        cps.append(pltpu.async_copy(ub_hbm.at[uidx_v.at[j]],
                                    ubv_v.at[dst], sem))
        cps.append(pltpu.async_copy(ib_hbm.at[iidx_v.at[j]],
                                    ibv_v.at[dst], sem))
    for cp in cps:
        cp.wait()

    iota16 = lax.iota(jnp.int32, 16)
    gb = gb_v[pl.ds(0, 16)]

    def group_body(g, carry):
        i0 = pl.multiple_of(g * 16, 16)
        row16 = i0 + iota16
        acc = ubv_v[pl.ds(i0, 16)] + ibv_v[pl.ds(i0, 16)] + gb
        for c in range(EMBED_DIM):
            cc = jnp.full((16,), c, jnp.int32)
            u = plsc.load_gather(urows_v, [row16, cc])
            v = plsc.load_gather(irows_v, [row16, cc])
            acc = acc + u * v
        out_v[pl.ds(i0, 16)] = acc
        return carry

    lax.fori_loop(0, GROUPS, group_body, 0)

    pltpu.sync_copy(out_v, out_hbm.at[pl.ds(base, BPW)])


def kernel(user_ids, item_ids, user_embedding, item_embedding, user_bias,
           item_bias, global_bias):
    uids = user_ids.astype(jnp.int32).reshape(NW, NCHUNK, CHUNK)
    iids = item_ids.astype(jnp.int32).reshape(NW, NCHUNK, CHUNK)
    ub = user_bias.reshape(-1)
    ib = item_bias.reshape(-1)
    gb = jnp.broadcast_to(global_bias.reshape(-1)[:1], (16,))
    return _mf_sc(uids, iids, user_embedding, item_embedding, ub, ib, gb)


# speedup vs baseline: 2.0844x; 2.0844x over previous
import functools

import jax
import jax.numpy as jnp
from jax import lax
from jax.experimental import pallas as pl
from jax.experimental.pallas import tpu as pltpu
from jax.experimental.pallas import tpu_sc as plsc

NUM_USERS = 1000000
NUM_ITEMS = 100000
EMBED_DIM = 32
BATCH = 16384

NC = 2
NS = 16
NW = NC * NS
BPW = BATCH // NW
C3 = 16
NCH = BPW // C3

_mesh = plsc.VectorSubcoreMesh(core_axis_name="c", subcore_axis_name="s")

_BLK = (C3, 8, EMBED_DIM)
_BBLK = (C3, 8)


@functools.partial(
    pl.kernel,
    out_type=jax.ShapeDtypeStruct((BATCH,), jnp.float32),
    mesh=_mesh,
    compiler_params=pltpu.CompilerParams(needs_layout_passes=False),
    scratch_types=[
        pltpu.VMEM((BPW,), jnp.int32),
        pltpu.VMEM((BPW,), jnp.int32),
        pltpu.VMEM((BPW,), jnp.int32),
        pltpu.VMEM((BPW,), jnp.int32),
        pltpu.VMEM(_BLK, jnp.float32),
        pltpu.VMEM(_BLK, jnp.float32),
        pltpu.VMEM(_BLK, jnp.float32),
        pltpu.VMEM(_BLK, jnp.float32),
        pltpu.VMEM(_BBLK, jnp.float32),
        pltpu.VMEM(_BBLK, jnp.float32),
        pltpu.VMEM(_BBLK, jnp.float32),
        pltpu.VMEM(_BBLK, jnp.float32),
        pltpu.VMEM((16,), jnp.float32),
        pltpu.VMEM((BPW,), jnp.float32),
        pltpu.SemaphoreType.DMA,
        pltpu.SemaphoreType.DMA,
    ],
)
def _mf_sc(uids_hbm, iids_hbm, utab_hbm, itab_hbm, ub_hbm, ib_hbm, gb_hbm,
           out_hbm, ublk_v, iblk_v, ur_v, ir_v, ur0, ur1, ir0, ir1, ubr0,
           ubr1, ibr0, ibr1, gb_v, out_v, sem0, sem1):
    wid = lax.axis_index("s") * NC + lax.axis_index("c")
    base = pl.multiple_of(wid * BPW, BPW)

    pltpu.sync_copy(uids_hbm.at[wid], ublk_v)
    pltpu.sync_copy(iids_hbm.at[wid], iblk_v)
    pltpu.sync_copy(gb_hbm, gb_v)

    iota16 = lax.iota(jnp.int32, 16)

    for k in range(BPW // 16):
        s = k * 16
        u = ublk_v[pl.ds(s, 16)]
        i = iblk_v[pl.ds(s, 16)]
        ur_v[pl.ds(s, 16)] = lax.bitwise_and(u, 7)
        ir_v[pl.ds(s, 16)] = lax.bitwise_and(i, 7)
        ublk_v[pl.ds(s, 16)] = lax.shift_right_logical(u, 3)
        iblk_v[pl.ds(s, 16)] = lax.shift_right_logical(i, 3)

    gb = gb_v[pl.ds(0, 16)]
    bufs = ((ur0, ir0, ubr0, ibr0, sem0), (ur1, ir1, ubr1, ibr1, sem1))

    def fire(ch, b):
        urows, irows, ubr, ibr, sem = bufs[b]
        ub16 = ublk_v[pl.ds(ch * C3, 16)]
        ib16 = iblk_v[pl.ds(ch * C3, 16)]
        for l in range(16):
            pltpu.async_copy(utab_hbm.at[ub16[l]], urows.at[l], sem)
            pltpu.async_copy(itab_hbm.at[ib16[l]], irows.at[l], sem)
            pltpu.async_copy(ub_hbm.at[ub16[l]], ubr.at[l], sem)
            pltpu.async_copy(ib_hbm.at[ib16[l]], ibr.at[l], sem)

    def drain(b):
        urows, irows, ubr, ibr, sem = bufs[b]
        pltpu.make_async_copy(utab_hbm.at[pl.ds(0, C3)], urows, sem).wait()
        pltpu.make_async_copy(itab_hbm.at[pl.ds(0, C3)], irows, sem).wait()
        pltpu.make_async_copy(ub_hbm.at[pl.ds(0, C3)], ubr, sem).wait()
        pltpu.make_async_copy(ib_hbm.at[pl.ds(0, C3)], ibr, sem).wait()

    def compute(ch, b):
        urows, irows, ubr, ibr, _ = bufs[b]
        i0 = pl.multiple_of(ch * C3, 16)
        ru = ur_v[pl.ds(i0, 16)]
        ri = ir_v[pl.ds(i0, 16)]
        acc = (plsc.load_gather(ubr, [iota16, ru])
               + plsc.load_gather(ibr, [iota16, ri]) + gb)
        for c in range(EMBED_DIM):
            cc = jnp.full((16,), c, jnp.int32)
            u = plsc.load_gather(urows, [iota16, ru, cc])
            v = plsc.load_gather(irows, [iota16, ri, cc])
            acc = acc + u * v
        out_v[pl.ds(i0, 16)] = acc

    fire(0, 0)

    def pair_body(h, carry):
        c0 = pl.multiple_of(h * 2, 2)
        fire(c0 + 1, 1)
        drain(0)
        compute(c0, 0)

        @pl.when(c0 + 2 < NCH)
        def _():
            fire(c0 + 2, 0)

        drain(1)
        compute(c0 + 1, 1)
        return carry

    lax.fori_loop(0, NCH // 2, pair_body, 0)

    pltpu.sync_copy(out_v, out_hbm.at[pl.ds(base, BPW)])


def kernel(user_ids, item_ids, user_embedding, item_embedding, user_bias,
           item_bias, global_bias):
    uids = user_ids.astype(jnp.int32).reshape(NW, BPW)
    iids = item_ids.astype(jnp.int32).reshape(NW, BPW)
    utab = user_embedding.reshape(NUM_USERS // 8, 8, EMBED_DIM)
    itab = item_embedding.reshape(NUM_ITEMS // 8, 8, EMBED_DIM)
    ub = user_bias.reshape(NUM_USERS // 8, 8)
    ib = item_bias.reshape(NUM_ITEMS // 8, 8)
    gb = jnp.broadcast_to(global_bias.reshape(-1)[:1], (16,))
    return _mf_sc(uids, iids, utab, itab, ub, ib, gb)
